# Initial kernel scaffold; baseline (speedup 1.0000x reference)
import jax
import jax.numpy as jnp
from jax.experimental import pallas as pl


def kernel(inputs, p):
    raise NotImplementedError("write your pallas kernel here")


# speedup vs baseline: 607.1435x; 607.1435x over previous
import math

import jax
import jax.numpy as jnp
import numpy as np
from jax.experimental import pallas as pl
from jax.experimental.pallas import tpu as pltpu

_N_BINS = 32
_DIM = 64
_RATIO = 1.2
_BOUND = 10.0
_BETA = 1e-06


def _mesh_consts_np():
    m = _N_BINS / 2
    x1l = _BOUND * (_RATIO - 1.0) / (math.pow(_RATIO, m) - 1.0)
    index = np.arange(0, _N_BINS + 1, dtype=np.float32).reshape(-1, 1) - m
    xr = (1.0 - np.power(_RATIO, np.abs(index))) / (1.0 - _RATIO)
    xr = np.where(index >= 0, x1l * xr, -x1l * xr).astype(np.float32)
    xr = (xr + _BOUND) / 2.0 / _BOUND
    mesh = np.concatenate(
        [np.zeros((1, 1), np.float32), xr[1:-1], np.ones((1, 1), np.float32)], 0
    )
    elmt = (mesh[1:] - mesh[:-1]).astype(np.float32)
    return mesh[:, 0], elmt[:, 0]


_MESH, _ELMT = _mesh_consts_np()


def _tables_kernel(p_ref, w_ref, e_ref, m_ref, c0_ref, c1_ref, c2_ref):
    w_half = w_ref[...]
    elmt_col = e_ref[...]
    m_col = m_ref[...]

    ep = jnp.exp(p_ref[...])
    denom = jnp.sum(ep * w_half, axis=0, keepdims=True)
    scale = (1.0 - (_ELMT[0] + _ELMT[-1]) * _BETA / 2.0) / denom
    px = ep * scale
    beta_row = jnp.full((1, _DIM), _BETA, jnp.float32)
    v1 = jnp.concatenate([beta_row, px], axis=0)
    v2 = jnp.concatenate([px, beta_row], axis=0)
    cell = (v1 + v2) * 0.5 * elmt_col

    rows = [jnp.zeros((1, _DIM), jnp.float32)]
    acc = jnp.zeros((1, _DIM), jnp.float32)
    for j in range(_N_BINS - 1):
        acc = acc + cell[j : j + 1]
        rows.append(acc)
    f_tab = jnp.concatenate(rows, axis=0)

    s = (v2 - v1) * (0.5 / elmt_col)
    c2_ref[...] = s
    c1_ref[...] = v1 - 2.0 * m_col * s
    c0_ref[...] = f_tab + (m_col * m_col) * s - m_col * v1


def _make_tables(p):
    out = jax.ShapeDtypeStruct((_N_BINS, _DIM), jnp.float32)
    w_half = jnp.asarray(((_ELMT[:-1] + _ELMT[1:]) / 2.0).reshape(-1, 1))
    elmt_col = jnp.asarray(_ELMT.reshape(-1, 1))
    m_col = jnp.asarray(_MESH[:_N_BINS].reshape(-1, 1))
    return pl.pallas_call(
        _tables_kernel,
        out_shape=(out, out, out),
    )(p, w_half, elmt_col, m_col)


_LANES = 128
_BLOCK_ROWS = 4096


def _map_kernel(x_ref, c0_ref, c1_ref, c2_ref, o_ref):
    u = x_ref[...]
    x = (u + _BOUND) / (2.0 * _BOUND)
    kf = jnp.full_like(x, -1.0)
    for j in range(_N_BINS + 1):
        kf = kf + jnp.where(x >= float(_MESH[j]), 1.0, 0.0)
    cover = jnp.logical_and(kf >= 0.0, kf <= float(_N_BINS - 1))
    kc = jnp.clip(kf, 0.0, float(_N_BINS - 1))

    c0 = jnp.zeros_like(x)
    c1 = jnp.zeros_like(x)
    c2 = jnp.zeros_like(x)
    for j in range(_N_BINS):
        m = kc == float(j)
        c0 = jnp.where(m, c0_ref[j, :], c0)
        c1 = jnp.where(m, c1_ref[j, :], c1)
        c2 = jnp.where(m, c2_ref[j, :], c2)

    yn = c0 + x * (c1 + x * c2)
    yn = jnp.where(cover, yn, x)
    y = yn * (2.0 * _BOUND) - _BOUND
    y = jnp.where(y > _BOUND, _BETA * (y - _BOUND) + _BOUND, y)
    y = jnp.where(y < -_BOUND, _BETA * (y + _BOUND) - _BOUND, y)
    o_ref[...] = y


def kernel(inputs, p):
    n, d = inputs.shape
    c0, c1, c2 = _make_tables(p)
    fold = _LANES // d
    x2 = inputs.reshape(n // fold, d * fold)
    tab_spec = pl.BlockSpec((_N_BINS, _LANES), lambda i: (0, 0))
    c0 = jnp.concatenate([c0] * fold, axis=1)
    c1 = jnp.concatenate([c1] * fold, axis=1)
    c2 = jnp.concatenate([c2] * fold, axis=1)
    rows = n // fold
    block_rows = min(_BLOCK_ROWS, rows)
    out = pl.pallas_call(
        _map_kernel,
        out_shape=jax.ShapeDtypeStruct(x2.shape, jnp.float32),
        grid=(rows // block_rows,),
        in_specs=[
            pl.BlockSpec((block_rows, _LANES), lambda i: (i, 0)),
            tab_spec,
            tab_spec,
            tab_spec,
        ],
        out_specs=pl.BlockSpec((block_rows, _LANES), lambda i: (i, 0)),
        compiler_params=pltpu.CompilerParams(
            dimension_semantics=("parallel",),
        ),
    )(x2, c0, c1, c2)
    return out.reshape(n, d)


# speedup vs baseline: 1192.6509x; 1.9644x over previous
import math

import jax
import jax.numpy as jnp
import numpy as np
from jax.experimental import pallas as pl
from jax.experimental.pallas import tpu as pltpu

_N_BINS = 32
_DIM = 64
_RATIO = 1.2
_BOUND = 10.0
_BETA = 1e-06


def _mesh_consts_np():
    m = _N_BINS / 2
    x1l = _BOUND * (_RATIO - 1.0) / (math.pow(_RATIO, m) - 1.0)
    index = np.arange(0, _N_BINS + 1, dtype=np.float32).reshape(-1, 1) - m
    xr = (1.0 - np.power(_RATIO, np.abs(index))) / (1.0 - _RATIO)
    xr = np.where(index >= 0, x1l * xr, -x1l * xr).astype(np.float32)
    xr = (xr + _BOUND) / 2.0 / _BOUND
    mesh = np.concatenate(
        [np.zeros((1, 1), np.float32), xr[1:-1], np.ones((1, 1), np.float32)], 0
    )
    elmt = (mesh[1:] - mesh[:-1]).astype(np.float32)
    return mesh[:, 0], elmt[:, 0]


_MESH, _ELMT = _mesh_consts_np()


def _tables_kernel(p_ref, w_ref, e_ref, m_ref, c0_ref, c1_ref, c2_ref):
    w_half = w_ref[...]
    elmt_col = e_ref[...]
    m_col = m_ref[...]

    ep = jnp.exp(p_ref[...])
    denom = jnp.sum(ep * w_half, axis=0, keepdims=True)
    scale = (1.0 - (_ELMT[0] + _ELMT[-1]) * _BETA / 2.0) / denom
    px = ep * scale
    beta_row = jnp.full((1, _DIM), _BETA, jnp.float32)
    v1 = jnp.concatenate([beta_row, px], axis=0)
    v2 = jnp.concatenate([px, beta_row], axis=0)
    cell = (v1 + v2) * 0.5 * elmt_col

    rows = [jnp.zeros((1, _DIM), jnp.float32)]
    acc = jnp.zeros((1, _DIM), jnp.float32)
    for j in range(_N_BINS - 1):
        acc = acc + cell[j : j + 1]
        rows.append(acc)
    f_tab = jnp.concatenate(rows, axis=0)

    s = (v2 - v1) * (0.5 / elmt_col)
    c2_ref[...] = s
    c1_ref[...] = v1 - 2.0 * m_col * s
    c0_ref[...] = f_tab + (m_col * m_col) * s - m_col * v1


def _make_tables(p):
    out = jax.ShapeDtypeStruct((_N_BINS, _DIM), jnp.float32)
    w_half = jnp.asarray(((_ELMT[:-1] + _ELMT[1:]) / 2.0).reshape(-1, 1))
    elmt_col = jnp.asarray(_ELMT.reshape(-1, 1))
    m_col = jnp.asarray(_MESH[:_N_BINS].reshape(-1, 1))
    return pl.pallas_call(
        _tables_kernel,
        out_shape=(out, out, out),
    )(p, w_half, elmt_col, m_col)


_LANES = 128
_BLOCK_ROWS = 4096


_X1L = _BOUND * (_RATIO - 1.0) / (math.pow(_RATIO, _N_BINS / 2) - 1.0)
_GA = (_RATIO - 1.0) / _X1L
_INV_LN_R = 1.0 / math.log(_RATIO)
_HALF = _N_BINS // 2


def _map_kernel(x_ref, c0_ref, c1_ref, c2_ref, o_ref):
    u = x_ref[...]
    x = (u + _BOUND) * (0.5 / _BOUND)
    g = jnp.log(1.0 + jnp.abs(u) * _GA) * _INV_LN_R
    f = jnp.floor(g)
    kf = jnp.where(u >= 0.0, float(_HALF) + f, float(_HALF - 1) - f)
    cover = jnp.logical_and(kf >= 0.0, kf <= float(_N_BINS - 1))
    kc = jnp.clip(kf, 0.0, float(_N_BINS - 1)).astype(jnp.int32)

    klo = jnp.bitwise_and(kc, 7)
    khi = jnp.right_shift(kc, 3)

    def _lut(tab_ref):
        parts = [
            jnp.take_along_axis(
                tab_ref[8 * i : 8 * (i + 1), :], klo, axis=0,
                mode="promise_in_bounds",
            )
            for i in range(4)
        ]
        a = jnp.where(khi == 1, parts[1], parts[0])
        b = jnp.where(khi == 3, parts[3], parts[2])
        return jnp.where(khi >= 2, b, a)

    c0 = _lut(c0_ref)
    c1 = _lut(c1_ref)
    c2 = _lut(c2_ref)

    yn = c0 + x * (c1 + x * c2)
    yn = jnp.where(cover, yn, x)
    y = yn * (2.0 * _BOUND) - _BOUND
    y = jnp.where(y > _BOUND, _BETA * (y - _BOUND) + _BOUND, y)
    y = jnp.where(y < -_BOUND, _BETA * (y + _BOUND) - _BOUND, y)
    o_ref[...] = y


def kernel(inputs, p):
    n, d = inputs.shape
    c0, c1, c2 = _make_tables(p)
    fold = _LANES // d
    x2 = inputs.reshape(n // fold, d * fold)
    tab_spec = pl.BlockSpec((_N_BINS, _LANES), lambda i: (0, 0))
    c0 = jnp.concatenate([c0] * fold, axis=1)
    c1 = jnp.concatenate([c1] * fold, axis=1)
    c2 = jnp.concatenate([c2] * fold, axis=1)
    rows = n // fold
    block_rows = min(_BLOCK_ROWS, rows)
    out = pl.pallas_call(
        _map_kernel,
        out_shape=jax.ShapeDtypeStruct(x2.shape, jnp.float32),
        grid=(rows // block_rows,),
        in_specs=[
            pl.BlockSpec((block_rows, _LANES), lambda i: (i, 0)),
            tab_spec,
            tab_spec,
            tab_spec,
        ],
        out_specs=pl.BlockSpec((block_rows, _LANES), lambda i: (i, 0)),
        compiler_params=pltpu.CompilerParams(
            dimension_semantics=("parallel",),
        ),
    )(x2, c0, c1, c2)
    return out.reshape(n, d)
